# baseline (speedup 1.0000x reference)
import jax
import jax.numpy as jnp
from jax.experimental import pallas as pl
from jax.experimental.pallas import tpu as pltpu

_MOMENTUM = 0.9


def _ema_scatter_body(tgt_ref, x_ref, mem_ref, out_ref):
    del tgt_ref
    out_ref[...] = _MOMENTUM * mem_ref[...] + (1.0 - _MOMENTUM) * x_ref[...]


def _update_mem(x, targets, mem):
    batch, _, feat = x.shape
    num_entries = mem.shape[0]
    vals = x[:, 0:1]
    mem3 = mem.reshape(num_entries, 1, feat)
    grid_spec = pltpu.PrefetchScalarGridSpec(
        num_scalar_prefetch=1,
        grid=(batch,),
        in_specs=[
            pl.BlockSpec((1, 1, feat), lambda i, tgt: (i, 0, 0)),
            pl.BlockSpec((1, 1, feat), lambda i, tgt: (tgt[i], 0, 0)),
        ],
        out_specs=pl.BlockSpec((1, 1, feat), lambda i, tgt: (tgt[i], 0, 0)),
    )
    out3 = pl.pallas_call(
        _ema_scatter_body,
        grid_spec=grid_spec,
        out_shape=jax.ShapeDtypeStruct((num_entries, 1, feat), jnp.float32),
        input_output_aliases={2: 0},
    )(targets, vals, mem3)
    return out3.reshape(num_entries, feat)


def kernel(x, targets, mem):
    new_mem = _update_mem(x, targets, mem)
    return x, new_mem


# speedup vs baseline: 2.2771x; 2.2771x over previous
import jax
import jax.numpy as jnp
from jax.experimental import pallas as pl
from jax.experimental.pallas import tpu as pltpu

_MOMENTUM = 0.9
_BLOCK_ROWS = 2000


def _copy_scatter_body(tgt_ref, vals_ref, mem_ref, out_ref):
    base = pl.program_id(0) * _BLOCK_ROWS
    out_ref[...] = mem_ref[...]
    n_upd = tgt_ref.shape[0]

    def upd(i, carry):
        t = tgt_ref[i]
        r = t - base

        @pl.when(jnp.logical_and(t >= base, t < base + _BLOCK_ROWS))
        def _():
            out_ref[pl.ds(r, 1), :] = (
                _MOMENTUM * mem_ref[pl.ds(r, 1), :]
                + (1.0 - _MOMENTUM) * vals_ref[pl.ds(i, 1), :]
            )

        return carry

    jax.lax.fori_loop(0, n_upd, upd, 0)


def _update_mem(x, targets, mem):
    batch, _, feat = x.shape
    num_entries = mem.shape[0]
    vals = x[:, 0]
    grid_spec = pltpu.PrefetchScalarGridSpec(
        num_scalar_prefetch=1,
        grid=(num_entries // _BLOCK_ROWS,),
        in_specs=[
            pl.BlockSpec((batch, feat), lambda i, tgt: (0, 0)),
            pl.BlockSpec((_BLOCK_ROWS, feat), lambda i, tgt: (i, 0)),
        ],
        out_specs=pl.BlockSpec((_BLOCK_ROWS, feat), lambda i, tgt: (i, 0)),
    )
    return pl.pallas_call(
        _copy_scatter_body,
        grid_spec=grid_spec,
        out_shape=jax.ShapeDtypeStruct((num_entries, feat), jnp.float32),
    )(targets, vals, mem)


def kernel(x, targets, mem):
    new_mem = _update_mem(x, targets, mem)
    return x, new_mem


# speedup vs baseline: 3.4949x; 1.5348x over previous
import jax
import jax.numpy as jnp
from jax.experimental import pallas as pl
from jax.experimental.pallas import tpu as pltpu

_MOMENTUM = 0.9
_BLOCK_ROWS = 2000


def _copy_scatter_body(bounds_ref, st_ref, perm_ref, vals_ref, mem_ref, out_ref):
    blk = pl.program_id(0)
    base = blk * _BLOCK_ROWS
    out_ref[...] = mem_ref[...]

    def upd(k, carry):
        r = st_ref[k] - base
        p = perm_ref[k]
        out_ref[pl.ds(r, 1), :] = (
            _MOMENTUM * mem_ref[pl.ds(r, 1), :]
            + (1.0 - _MOMENTUM) * vals_ref[pl.ds(p, 1), :]
        )
        return carry

    jax.lax.fori_loop(bounds_ref[blk], bounds_ref[blk + 1], upd, 0)


def _update_mem(x, targets, mem):
    batch, _, feat = x.shape
    num_entries = mem.shape[0]
    num_blocks = num_entries // _BLOCK_ROWS
    vals = x[:, 0]
    perm = jnp.argsort(targets, stable=True)
    st = jnp.take(targets, perm)
    block_edges = jnp.arange(0, num_entries + _BLOCK_ROWS, _BLOCK_ROWS,
                             dtype=jnp.int32)
    bounds = jnp.searchsorted(st, block_edges, side="left").astype(jnp.int32)
    grid_spec = pltpu.PrefetchScalarGridSpec(
        num_scalar_prefetch=3,
        grid=(num_blocks,),
        in_specs=[
            pl.BlockSpec((batch, feat), lambda i, *_: (0, 0)),
            pl.BlockSpec((_BLOCK_ROWS, feat), lambda i, *_: (i, 0)),
        ],
        out_specs=pl.BlockSpec((_BLOCK_ROWS, feat), lambda i, *_: (i, 0)),
    )
    return pl.pallas_call(
        _copy_scatter_body,
        grid_spec=grid_spec,
        out_shape=jax.ShapeDtypeStruct((num_entries, feat), jnp.float32),
    )(bounds, st, perm.astype(jnp.int32), vals, mem)


def kernel(x, targets, mem):
    new_mem = _update_mem(x, targets, mem)
    return x, new_mem
